# Initial kernel scaffold; baseline (speedup 1.0000x reference)
import jax
import jax.numpy as jnp
from jax.experimental import pallas as pl


def kernel(prediction, gt_tensor):
    raise NotImplementedError("write your pallas kernel here")


# speedup vs baseline: 3.4047x; 3.4047x over previous
import jax
import jax.numpy as jnp
from jax.experimental import pallas as pl
from jax.experimental.pallas import tpu as pltpu

_S = 7.0
_M = 6272
_R = 49


def _loss_body(p_ref, g_ref, out_ref):
    p = p_ref[...]
    g = g_ref[...]
    obj = (g[4] > 0).astype(jnp.float32)
    noobj = 1.0 - obj

    d4 = p[4] - g[4]
    d9 = p[9] - g[9]
    loss_noobj = jnp.sum((d4 * d4 + d9 * d9) * noobj)

    dc = p[10:30] - g[10:30]
    loss_class = jnp.sum(dc * dc * obj[None])

    gx, gy, gw, gh = g[0], g[1], g[2], g[3]
    tx1 = gx / _S - gw * 0.5
    ty1 = gy / _S - gh * 0.5
    tx2 = gx / _S + gw * 0.5
    ty2 = gy / _S + gh * 0.5
    area_t = (tx2 - tx1) * (ty2 - ty1)

    def iou_for(off):
        x, y, w, h = p[off], p[off + 1], p[off + 2], p[off + 3]
        x1 = x / _S - w * 0.5
        y1 = y / _S - h * 0.5
        x2 = x / _S + w * 0.5
        y2 = y / _S + h * 0.5
        iw = jnp.clip(jnp.minimum(x2, tx2) - jnp.maximum(x1, tx1), 0.0)
        ih = jnp.clip(jnp.minimum(y2, ty2) - jnp.maximum(y1, ty1), 0.0)
        inter = iw * ih
        area_p = (x2 - x1) * (y2 - y1)
        return inter / (area_p + area_t - inter)

    iou0 = iou_for(0)
    iou1 = iou_for(5)
    sel = iou1 > iou0
    max_iou = jnp.maximum(iou0, iou1)

    def pick(a0, a1):
        return jnp.where(sel, a1, a0)

    prx = pick(p[0], p[5])
    pry = pick(p[1], p[6])
    prw = pick(p[2], p[7])
    prh = pick(p[3], p[8])
    prc = pick(p[4], p[9])
    trx = pick(g[0], g[5])
    try_ = pick(g[1], g[6])
    trw = pick(g[2], g[7])
    trh = pick(g[3], g[8])

    dx = prx - trx
    dy = pry - try_
    loss_xy = jnp.sum((dx * dx + dy * dy) * obj)
    dw = jnp.sqrt(prw) - jnp.sqrt(trw)
    dh = jnp.sqrt(prh) - jnp.sqrt(trh)
    loss_wh = jnp.sum((dw * dw + dh * dh) * obj)
    do = prc - max_iou
    loss_obj = jnp.sum(do * do * obj)

    total = 5.0 * (loss_xy + loss_wh) + loss_obj + 0.5 * loss_noobj + loss_class
    out_ref[0, 0] = total * (1.0 / 128.0)


def kernel(prediction, gt_tensor):
    p = prediction.reshape(_M, 30).T.reshape(30, _R, 128)
    g = gt_tensor.reshape(_M, 30).T.reshape(30, _R, 128)
    out = pl.pallas_call(
        _loss_body,
        out_shape=jax.ShapeDtypeStruct((1, 1), jnp.float32),
        in_specs=[
            pl.BlockSpec(memory_space=pltpu.VMEM),
            pl.BlockSpec(memory_space=pltpu.VMEM),
        ],
        out_specs=pl.BlockSpec(memory_space=pltpu.SMEM),
    )(p, g)
    return out[0, 0]
